# Initial kernel scaffold; baseline (speedup 1.0000x reference)
import jax
import jax.numpy as jnp
from jax.experimental import pallas as pl


def kernel(x, W, b):
    raise NotImplementedError("write your pallas kernel here")


# speedup vs baseline: 1.1645x; 1.1645x over previous
import functools

import jax
import jax.numpy as jnp
from jax.experimental import pallas as pl

TOPK = 8
NUM_EXPERTS = 64
BR = 512


def _router_block(x_ref, wt_ref, b_ref, w_out_ref, i_out_ref):
    xb = x_ref[...]
    wt = wt_ref[...]
    logits = jax.lax.dot_general(
        xb, wt, dimension_numbers=(((1,), (0,)), ((), ())),
        preferred_element_type=jnp.float32,
    )
    logits = logits + b_ref[...]

    iota = jax.lax.broadcasted_iota(jnp.int32, (BR, NUM_EXPERTS), 1)
    neg_inf = jnp.float32(-jnp.inf)
    vals = []
    idxs = []
    l = logits
    for _ in range(TOPK):
        m = jnp.max(l, axis=1, keepdims=True)
        am = jnp.min(jnp.where(l == m, iota, NUM_EXPERTS), axis=1, keepdims=True)
        vals.append(m)
        idxs.append(am)
        l = jnp.where(iota == am, neg_inf, l)

    v = jnp.concatenate(vals, axis=1)
    e = jnp.exp(v - vals[0])
    w = e / jnp.sum(e, axis=1, keepdims=True)
    w_out_ref[...] = w
    i_out_ref[...] = jnp.concatenate(idxs, axis=1)


@functools.partial(jax.jit, static_argnames=())
def kernel(x, W, b):
    n_rows, d = x.shape
    wt = W.T
    b2 = b.reshape(1, NUM_EXPERTS)
    grid = (n_rows // BR,)
    w_out, i_out = pl.pallas_call(
        _router_block,
        grid=grid,
        in_specs=[
            pl.BlockSpec((BR, d), lambda i: (i, 0)),
            pl.BlockSpec((d, NUM_EXPERTS), lambda i: (0, 0)),
            pl.BlockSpec((1, NUM_EXPERTS), lambda i: (0, 0)),
        ],
        out_specs=[
            pl.BlockSpec((BR, TOPK), lambda i: (i, 0)),
            pl.BlockSpec((BR, TOPK), lambda i: (i, 0)),
        ],
        out_shape=[
            jax.ShapeDtypeStruct((n_rows, TOPK), jnp.float32),
            jax.ShapeDtypeStruct((n_rows, TOPK), jnp.int32),
        ],
    )(x, wt, b2)
    return (w_out, i_out)


# speedup vs baseline: 1.2614x; 1.0832x over previous
import functools

import jax
import jax.numpy as jnp
from jax.experimental import pallas as pl

TOPK = 8
NUM_EXPERTS = 64
BR = 512


def _router_block(x_ref, wt_ref, b_ref, w_out_ref, i_out_ref):
    xb = x_ref[...]
    wt = wt_ref[...]
    logits = jax.lax.dot_general(
        xb, wt, dimension_numbers=(((1,), (0,)), ((), ())),
        preferred_element_type=jnp.float32,
    )
    logits = logits + b_ref[...]

    s = jax.lax.bitcast_convert_type(logits, jnp.int32)
    sortable = jnp.where(s < 0, s ^ jnp.int32(0x7FFFFFFF), s)
    iota_rev = (NUM_EXPERTS - 1) - jax.lax.broadcasted_iota(
        jnp.int32, (BR, NUM_EXPERTS), 1)
    keys = (sortable & jnp.int32(~63)) | iota_rev

    neg_key = jnp.int32(-(2**31))
    vals = []
    idxs = []
    for _ in range(TOPK):
        m = jnp.max(keys, axis=1, keepdims=True)
        keys = jnp.where(keys == m, neg_key, keys)
        vt = m & jnp.int32(~63)
        vs = jnp.where(vt < 0, vt ^ jnp.int32(0x7FFFFFFF), vt)
        vals.append(jax.lax.bitcast_convert_type(vs, jnp.float32))
        idxs.append((NUM_EXPERTS - 1) - (m & jnp.int32(63)))

    v = jnp.concatenate(vals, axis=1)
    e = jnp.exp(v - vals[0])
    w = e / jnp.sum(e, axis=1, keepdims=True)
    w_out_ref[...] = w
    i_out_ref[...] = jnp.concatenate(idxs, axis=1)


@functools.partial(jax.jit, static_argnames=())
def kernel(x, W, b):
    n_rows, d = x.shape
    wt = W.T
    b2 = b.reshape(1, NUM_EXPERTS)
    grid = (n_rows // BR,)
    w_out, i_out = pl.pallas_call(
        _router_block,
        grid=grid,
        in_specs=[
            pl.BlockSpec((BR, d), lambda i: (i, 0)),
            pl.BlockSpec((d, NUM_EXPERTS), lambda i: (0, 0)),
            pl.BlockSpec((1, NUM_EXPERTS), lambda i: (0, 0)),
        ],
        out_specs=[
            pl.BlockSpec((BR, TOPK), lambda i: (i, 0)),
            pl.BlockSpec((BR, TOPK), lambda i: (i, 0)),
        ],
        out_shape=[
            jax.ShapeDtypeStruct((n_rows, TOPK), jnp.float32),
            jax.ShapeDtypeStruct((n_rows, TOPK), jnp.int32),
        ],
    )(x, wt, b2)
    return (w_out, i_out)


# speedup vs baseline: 1.4864x; 1.1784x over previous
import functools

import jax
import jax.numpy as jnp
from jax.experimental import pallas as pl

TOPK = 8
NUM_EXPERTS = 64
BR = 512


def _router_block(x_ref, wt_ref, b_ref, w_out_ref, i_out_ref):
    xb = x_ref[...]
    wt = wt_ref[...]
    logits = jax.lax.dot_general(
        xb, wt, dimension_numbers=(((1,), (0,)), ((), ())),
        preferred_element_type=jnp.float32,
    )
    logits = logits + b_ref[...]

    lt = logits.T

    s = jax.lax.bitcast_convert_type(lt, jnp.int32)
    sortable = jnp.where(s < 0, s ^ jnp.int32(0x7FFFFFFF), s)
    iota_rev = (NUM_EXPERTS - 1) - jax.lax.broadcasted_iota(
        jnp.int32, (NUM_EXPERTS, BR), 0)
    keys = (sortable & jnp.int32(~63)) | iota_rev

    neg_key = jnp.int32(-(2**31))
    vals = []
    idxs = []
    for _ in range(TOPK):
        m = jnp.max(keys, axis=0, keepdims=True)
        keys = jnp.where(keys == m, neg_key, keys)
        vt = m & jnp.int32(~63)
        vs = jnp.where(vt < 0, vt ^ jnp.int32(0x7FFFFFFF), vt)
        vals.append(jax.lax.bitcast_convert_type(vs, jnp.float32))
        idxs.append((NUM_EXPERTS - 1) - (m & jnp.int32(63)))

    v = jnp.concatenate(vals, axis=0)
    e = jnp.exp(v - vals[0])
    w = e / jnp.sum(e, axis=0, keepdims=True)
    w_out_ref[...] = w.T
    i_out_ref[...] = jnp.concatenate(idxs, axis=0).T


@functools.partial(jax.jit, static_argnames=())
def kernel(x, W, b):
    n_rows, d = x.shape
    wt = W.T
    b2 = b.reshape(1, NUM_EXPERTS)
    grid = (n_rows // BR,)
    w_out, i_out = pl.pallas_call(
        _router_block,
        grid=grid,
        in_specs=[
            pl.BlockSpec((BR, d), lambda i: (i, 0)),
            pl.BlockSpec((d, NUM_EXPERTS), lambda i: (0, 0)),
            pl.BlockSpec((1, NUM_EXPERTS), lambda i: (0, 0)),
        ],
        out_specs=[
            pl.BlockSpec((BR, TOPK), lambda i: (i, 0)),
            pl.BlockSpec((BR, TOPK), lambda i: (i, 0)),
        ],
        out_shape=[
            jax.ShapeDtypeStruct((n_rows, TOPK), jnp.float32),
            jax.ShapeDtypeStruct((n_rows, TOPK), jnp.int32),
        ],
    )(x, wt, b2)
    return (w_out, i_out)
